# baseline (speedup 1.0000x reference)
import functools

import jax
import jax.numpy as jnp
from jax import lax
from jax.experimental import pallas as pl
from jax.experimental.pallas import tpu as pltpu
from jax.experimental.pallas import tpu_sc as plsc

B = 16384
F = 3
VOCAB = 1000000
EMB = 16

NW = 32
BPW = B // NW
RPW = F * BPW
NCH = RPW // 128
TPW = RPW // 16
GPW = BPW // 16

_mesh = plsc.VectorSubcoreMesh(core_axis_name="c", subcore_axis_name="s")
_params = pltpu.CompilerParams(needs_layout_passes=False,
                               use_tc_tiling_on_sc=False)


def _iota16():
    return lax.iota(jnp.int32, 16)


def _splat(x):
    return jnp.broadcast_to(jnp.asarray(x, jnp.int32), (16,))


def _bcast_lane(vec, c):
    dn = lax.GatherDimensionNumbers(offset_dims=(), collapsed_slice_dims=(0,),
                                    start_index_map=(0,))
    return lax.gather(vec, _splat(c)[:, None], dn, (1,),
                      mode=lax.GatherScatterMode.PROMISE_IN_BOUNDS)


@functools.partial(
    pl.kernel,
    out_type=(
        jax.ShapeDtypeStruct((F * B,), jnp.float32),
        jax.ShapeDtypeStruct((F * B,), jnp.float32),
    ),
    mesh=_mesh,
    compiler_params=_params,
    scratch_types=[
        pltpu.VMEM((NCH, 128), jnp.int32),
        pltpu.VMEM((RPW, EMB), jnp.float32),
        pltpu.VMEM((RPW,), jnp.float32),
        pltpu.VMEM(((F + 15) * 16,), jnp.float32),
        pltpu.VMEM((RPW,), jnp.float32),
        pltpu.VMEM((RPW,), jnp.float32),
        pltpu.SemaphoreType.DMA,
    ],
)
def _attn_terms(so_hbm, idx_hbm, xv_hbm, par_hbm, t1_hbm, t2_hbm,
                idx_v, rows_v, xv_v, par_v, t1loc, t2loc, sem):
    w = lax.axis_index("s") * 2 + lax.axis_index("c")

    pltpu.sync_copy(idx_hbm.at[w], idx_v)
    copies = []
    for j in range(NCH):
        copies.append(pltpu.async_copy(
            so_hbm.at[idx_v.at[j]], rows_v.at[pl.ds(j * 128, 128)], sem))
    pltpu.sync_copy(xv_hbm.at[w], xv_v)
    pltpu.sync_copy(par_hbm, par_v)

    v_acc = jnp.zeros((16,), jnp.float32)
    for k in range(16):
        hk = plsc.load_gather(par_v, [_splat(256 + k)])
        v_acc = v_acc + hk * par_v[pl.ds(k * 16, 16)]
    p_vec = par_v[pl.ds(272, 16)]

    for c in copies:
        c.wait()

    lane96 = _iota16() * 96

    def tile_body(ft, _):
        xv = xv_v[pl.ds(ft * 16, 16)]
        xv2 = xv * xv
        u1 = xv2 * v_acc
        u2 = xv2 * p_vec
        acc1 = jnp.zeros((16,), jnp.float32)
        acc2 = jnp.zeros((16,), jnp.float32)
        for c in range(16):
            row = rows_v[ft * 16 + c, :]
            rsq = row * row
            acc1 = acc1 + _bcast_lane(u1, c) * rsq
            acc2 = acc2 + _bcast_lane(u2, c) * rsq
        idx = lane96 + _splat(ft)
        plsc.store_scatter(t1loc, [idx], acc1)
        plsc.store_scatter(t2loc, [idx], acc2)
        return _

    lax.fori_loop(0, TPW, tile_body, None)

    for e in range(16):
        for f in range(F):
            src = pl.ds(e * 96 + f * 32, 32)
            dst = pl.ds(e * 3072 + f * 1024 + w * 32, 32)
            pltpu.sync_copy(t1loc.at[src], t1_hbm.at[dst])
            pltpu.sync_copy(t2loc.at[src], t2_hbm.at[dst])


@functools.partial(
    pl.kernel,
    out_type=jax.ShapeDtypeStruct((B,), jnp.float32),
    mesh=_mesh,
    compiler_params=_params,
    scratch_types=[
        pltpu.VMEM((NCH, 128), jnp.int32),
        pltpu.VMEM((RPW,), jnp.float32),
        pltpu.VMEM((RPW,), jnp.float32),
        pltpu.VMEM((RPW,), jnp.float32),
        pltpu.VMEM((RPW,), jnp.float32),
        pltpu.VMEM((16,), jnp.float32),
        pltpu.VMEM((BPW,), jnp.float32),
        pltpu.SemaphoreType.DMA,
    ],
)
def _combine(fo_hbm, idx_hbm, xv_hbm, t1f_hbm, t2f_hbm, bias_hbm, out_hbm,
             idx_v, fo_v, t1_v, t2_v, xv_v, bias_v, out_loc, sem):
    w = lax.axis_index("s") * 2 + lax.axis_index("c")

    pltpu.sync_copy(idx_hbm.at[w], idx_v)
    copies = []
    for j in range(NCH):
        copies.append(pltpu.async_copy(
            fo_hbm.at[idx_v.at[j]], fo_v.at[pl.ds(j * 128, 128)], sem))
    pltpu.sync_copy(t1f_hbm.at[pl.ds(w * RPW, RPW)], t1_v)
    pltpu.sync_copy(t2f_hbm.at[pl.ds(w * RPW, RPW)], t2_v)
    pltpu.sync_copy(xv_hbm.at[w], xv_v)
    pltpu.sync_copy(bias_hbm, bias_v)

    bias_vec = bias_v[...]
    lane3 = _iota16() * 3

    for c in copies:
        c.wait()

    def out_body(g, _):
        base = g * 16
        acc = bias_vec
        for f in range(F):
            acc = acc + fo_v[pl.ds(f * BPW + base, 16)] * \
                xv_v[pl.ds(f * BPW + base, 16)]
        i0 = _splat(base * 3) + lane3
        a0 = plsc.load_gather(t1_v, [i0])
        a1 = plsc.load_gather(t1_v, [i0 + 1])
        a2 = plsc.load_gather(t1_v, [i0 + 2])
        b0 = plsc.load_gather(t2_v, [i0])
        b1 = plsc.load_gather(t2_v, [i0 + 1])
        b2 = plsc.load_gather(t2_v, [i0 + 2])
        mx = jnp.maximum(a0, jnp.maximum(a1, a2))
        e0 = jnp.exp(a0 - mx)
        e1 = jnp.exp(a1 - mx)
        e2 = jnp.exp(a2 - mx)
        num = e0 * b0 + e1 * b1 + e2 * b2
        den = e0 + e1 + e2
        out_loc[pl.ds(base, 16)] = acc + num / den
        return _

    lax.fori_loop(0, GPW, out_body, None)

    pltpu.sync_copy(out_loc, out_hbm.at[pl.ds(w * BPW, BPW)])


def kernel(Xi, Xv, fo_tables, so_tables, W_att, b_att, H, P, bias):
    so_flat = so_tables.reshape(F * VOCAB, EMB)
    fo_flat = fo_tables.reshape(F * VOCAB)

    idx = (Xi.T.astype(jnp.int32)
           + (jnp.arange(F, dtype=jnp.int32) * VOCAB)[:, None])
    idx_w = idx.reshape(F, NW, BPW).transpose(1, 0, 2).reshape(NW, NCH, 128)
    xv_w = Xv.T.reshape(F, NW, BPW).transpose(1, 0, 2).reshape(NW, RPW)

    params = jnp.concatenate(
        [W_att.T, H[None, :], P[None, :]], axis=0).astype(jnp.float32)
    bias16 = jnp.broadcast_to(bias.astype(jnp.float32), (16,))

    t1, t2 = _attn_terms(so_flat, idx_w, xv_w, params.reshape(-1))
    return _combine(fo_flat, idx_w, xv_w, t1, t2, bias16)
